# baseline (speedup 1.0000x reference)
import functools

import jax
import jax.numpy as jnp
from jax import lax
from jax.experimental import pallas as pl
from jax.experimental.pallas import tpu as pltpu
from jax.experimental.pallas import tpu_sc as plsc

_NW = 32
_RPC = 8
_LANES = 16


def _sc_body(pe_hbm, out_hbm, pe_v, buf0, buf1, sem0, sem1, *, flat, batch):
    wid = lax.axis_index("s") * 2 + lax.axis_index("c")
    pltpu.sync_copy(pe_hbm, pe_v)
    n_chunks = flat // _RPC
    per_w = n_chunks // _NW
    base = wid * per_w

    def fill(buf, c):
        for k in range(_RPC):
            idx = jnp.full((_LANES,), c * _RPC + k, jnp.int32)
            val = plsc.load_gather(pe_v, [idx])

            def body(j, _):
                for u in range(8):
                    buf[k, pl.ds((j * 8 + u) * _LANES, _LANES)] = val
                return 0

            lax.fori_loop(0, batch // (8 * _LANES), body, 0)

    def step(t, _):
        c0 = base + 2 * t
        c1 = c0 + 1

        @pl.when(t > 0)
        def _():
            pltpu.make_async_copy(buf0, out_hbm.at[pl.ds(c0 * _RPC, _RPC), :], sem0).wait()
            pltpu.make_async_copy(buf1, out_hbm.at[pl.ds(c1 * _RPC, _RPC), :], sem1).wait()

        fill(buf0, c0)
        pltpu.make_async_copy(buf0, out_hbm.at[pl.ds(c0 * _RPC, _RPC), :], sem0).start()
        fill(buf1, c1)
        pltpu.make_async_copy(buf1, out_hbm.at[pl.ds(c1 * _RPC, _RPC), :], sem1).start()
        return 0

    lax.fori_loop(0, per_w // 2, step, 0)
    last0 = base + per_w - 2
    pltpu.make_async_copy(buf0, out_hbm.at[pl.ds(last0 * _RPC, _RPC), :], sem0).wait()
    pltpu.make_async_copy(buf1, out_hbm.at[pl.ds((last0 + 1) * _RPC, _RPC), :], sem1).wait()


def kernel(x, pe_weight):
    batch = x.shape[0]
    max_len, d_model = pe_weight.shape
    flat = max_len * d_model
    mesh = plsc.VectorSubcoreMesh(core_axis_name="c", subcore_axis_name="s")
    k = pl.kernel(
        functools.partial(_sc_body, flat=flat, batch=batch),
        out_type=jax.ShapeDtypeStruct((flat, batch), pe_weight.dtype),
        mesh=mesh,
        scratch_types=[
            pltpu.VMEM((flat,), pe_weight.dtype),
            pltpu.VMEM((_RPC, batch), pe_weight.dtype),
            pltpu.VMEM((_RPC, batch), pe_weight.dtype),
            pltpu.SemaphoreType.DMA,
            pltpu.SemaphoreType.DMA,
        ],
        compiler_params=pltpu.CompilerParams(
            use_tc_tiling_on_sc=True, needs_layout_passes=False
        ),
    )
    out_t = k(pe_weight.reshape(flat))
    return out_t.T.reshape(batch, max_len, d_model)


# speedup vs baseline: 1.2925x; 1.2925x over previous
import functools

import jax
import jax.numpy as jnp
from jax.experimental import pallas as pl


def _bcast_kernel(pe_ref, out_ref, *, pb):
    i = pl.program_id(0)
    for p in range(pb):
        row = pe_ref[pl.ds(i * pb + p, 1), :]
        col = jnp.swapaxes(row, 0, 1)
        out_ref[p, :, :] = jnp.broadcast_to(col, out_ref.shape[1:])


def kernel(x, pe_weight):
    batch = x.shape[0]
    max_len, d_model = pe_weight.shape
    pb = 2
    out3 = pl.pallas_call(
        functools.partial(_bcast_kernel, pb=pb),
        grid=(max_len // pb,),
        in_specs=[pl.BlockSpec((max_len, d_model), lambda i: (0, 0))],
        out_specs=pl.BlockSpec((pb, d_model, batch), lambda i: (i, 0, 0)),
        out_shape=jax.ShapeDtypeStruct((max_len, d_model, batch), pe_weight.dtype),
    )(pe_weight)
    return out3.transpose(2, 0, 1)


# speedup vs baseline: 1.6337x; 1.2640x over previous
import functools

import jax
import jax.numpy as jnp
from jax.experimental import pallas as pl
from jax.experimental.pallas import tpu as pltpu


def _bcast_kernel(pe_hbm, out_ref, pe_v, sem, *, pb):
    i = pl.program_id(0)

    @pl.when(i == 0)
    def _():
        pltpu.make_async_copy(pe_hbm, pe_v, sem).start()
        pltpu.make_async_copy(pe_hbm, pe_v, sem).wait()

    for p in range(pb):
        row = pe_v[pl.ds(i * pb + p, 1), :]
        col = jnp.swapaxes(row, 0, 1)
        out_ref[p, :, :] = jnp.broadcast_to(col, out_ref.shape[1:])


def kernel(x, pe_weight):
    batch = x.shape[0]
    max_len, d_model = pe_weight.shape
    pb = 4
    out3 = pl.pallas_call(
        functools.partial(_bcast_kernel, pb=pb),
        grid=(max_len // pb,),
        in_specs=[pl.BlockSpec(memory_space=pl.ANY)],
        out_specs=pl.BlockSpec((pb, d_model, batch), lambda i: (i, 0, 0)),
        out_shape=jax.ShapeDtypeStruct((max_len, d_model, batch), pe_weight.dtype),
        scratch_shapes=[
            pltpu.VMEM((max_len, d_model), pe_weight.dtype),
            pltpu.SemaphoreType.DMA,
        ],
    )(pe_weight)
    return out3.transpose(2, 0, 1)


# speedup vs baseline: 1.6353x; 1.0010x over previous
import functools

import jax
import jax.numpy as jnp
from jax.experimental import pallas as pl
from jax.experimental.pallas import tpu as pltpu


def _bcast_kernel(pe_hbm, out_ref, pe_v, sem, *, pb):
    i = pl.program_id(0)

    @pl.when(i == 0)
    def _():
        pltpu.make_async_copy(pe_hbm, pe_v, sem).start()
        pltpu.make_async_copy(pe_hbm, pe_v, sem).wait()

    for p in range(pb):
        row = pe_v[pl.ds(i * pb + p, 1), :]
        col = jnp.swapaxes(row, 0, 1)
        out_ref[p, :, :] = jnp.broadcast_to(col, out_ref.shape[1:])


def kernel(x, pe_weight):
    batch = x.shape[0]
    max_len, d_model = pe_weight.shape
    pb = 5
    out3 = pl.pallas_call(
        functools.partial(_bcast_kernel, pb=pb),
        grid=(max_len // pb,),
        in_specs=[pl.BlockSpec(memory_space=pl.ANY)],
        out_specs=pl.BlockSpec((pb, d_model, batch), lambda i: (i, 0, 0)),
        out_shape=jax.ShapeDtypeStruct((max_len, d_model, batch), pe_weight.dtype),
        scratch_shapes=[
            pltpu.VMEM((max_len, d_model), pe_weight.dtype),
            pltpu.SemaphoreType.DMA,
        ],
    )(pe_weight)
    return out3.transpose(2, 0, 1)


# speedup vs baseline: 1.6365x; 1.0008x over previous
import functools

import jax
import jax.numpy as jnp
from jax.experimental import pallas as pl
from jax.experimental.pallas import tpu as pltpu


def _bcast_kernel(pe_hbm, out_ref, pe_v, sem, *, pb):
    i = pl.program_id(0)

    @pl.when(i == 0)
    def _():
        pltpu.make_async_copy(pe_hbm, pe_v, sem).start()
        pltpu.make_async_copy(pe_hbm, pe_v, sem).wait()

    for p in range(pb):
        row = pe_v[pl.ds(i * pb + p, 1), :]
        col = jnp.swapaxes(row, 0, 1)
        out_ref[p, :, :] = jnp.broadcast_to(col, out_ref.shape[1:])


def kernel(x, pe_weight):
    batch = x.shape[0]
    max_len, d_model = pe_weight.shape
    pb = 4
    out3 = pl.pallas_call(
        functools.partial(_bcast_kernel, pb=pb),
        grid=(max_len // pb,),
        in_specs=[pl.BlockSpec(memory_space=pl.ANY)],
        out_specs=pl.BlockSpec((pb, d_model, batch), lambda i: (i, 0, 0)),
        out_shape=jax.ShapeDtypeStruct((max_len, d_model, batch), pe_weight.dtype),
        scratch_shapes=[
            pltpu.VMEM((max_len, d_model), pe_weight.dtype),
            pltpu.SemaphoreType.DMA,
        ],
    )(pe_weight)
    return out3.transpose(2, 0, 1)
